# baseline (speedup 1.0000x reference)
import functools

import jax
import jax.numpy as jnp
from jax import lax
from jax.experimental import pallas as pl
from jax.experimental.pallas import tpu as pltpu
from jax.experimental.pallas import tpu_sc as plsc

N_NODES = 10000
N_EDGES = 320000
A = 128
L = 16
NP = A // 2
CP = 4
NCHUNK = NP // CP
NC = 2
NS = 16
NW = NC * NS
E_PER = N_EDGES // NW
NG = E_PER // L


def _body(ct_hbm, sidx_hbm, didx_hbm, w_hbm, out_hbm,
          si_v, di_v, pmax, wv, sl0, sl1, sem0, sem1):
  cid = lax.axis_index("c")
  sid = lax.axis_index("s")
  wid = sid * NC + cid
  base = wid * E_PER

  pltpu.sync_copy(sidx_hbm.at[pl.ds(base, E_PER)], si_v)
  pltpu.sync_copy(didx_hbm.at[pl.ds(base, E_PER)], di_v)
  pltpu.sync_copy(w_hbm, wv)

  slab = (sl0, sl1)
  sem = (sem0, sem1)

  def start_chunk(c, b):
    pltpu.async_copy(ct_hbm.at[pl.ds(c * CP, CP)], slab[b], sem[b])

  def wait_chunk(c, b):
    pltpu.make_async_copy(ct_hbm.at[pl.ds(c * CP, CP)], slab[b], sem[b]).wait()

  start_chunk(0, 0)
  start_chunk(1, 1)

  for c in range(NCHUNK):
    b = c % 2
    wait_chunk(c, b)
    if c + 2 < NCHUNK:
      start_chunk(c + 2, b)
    sl = slab[b]
    wwin = wv[pl.ds((c * CP // L) * L, L)]
    woff = c * CP - (c * CP // L) * L
    wp = [
        plsc.bitcast(jnp.full((L,), wwin[woff + jj], jnp.int32),
                     jnp.bfloat16)
        for jj in range(CP)
    ]
    first = c == 0

    @pl.loop(0, NG, unroll=5)
    def _grp(grp, sl=sl, wp=wp, first=first):
      sv = si_v[pl.ds(grp * L, L)]
      dv = di_v[pl.ds(grp * L, L)]
      accp = None
      for jj in range(CP):
        row = sl.at[jj]
        s = plsc.bitcast(plsc.load_gather(row, [sv]), jnp.bfloat16)
        d = plsc.bitcast(plsc.load_gather(row, [dv]), jnp.bfloat16)
        m = s * d * wp[jj]
        accp = m if jj == 0 else jnp.maximum(accp, m)
      lo, hi = plsc.unpack(
          accp, format=plsc.PackFormat.INTERLEAVED,
          preferred_element_type=jnp.float32)
      cm = jnp.maximum(lo, hi)
      if not first:
        cm = jnp.maximum(cm, pmax[pl.ds(grp * L, L)])
      pmax[pl.ds(grp * L, L)] = cm

  @pl.loop(0, NG, unroll=5)
  def _sig(i):
    x = pmax[pl.ds(i * L, L)]
    pmax[pl.ds(i * L, L)] = 1.0 / (1.0 + jnp.exp(-x))

  pltpu.sync_copy(pmax, out_hbm.at[pl.ds(base, E_PER)])


@jax.jit
def _run(ct, sidx, didx, w):
  mesh = plsc.VectorSubcoreMesh(
      core_axis_name="c", subcore_axis_name="s", num_cores=NC,
      num_subcores=NS)
  f = pl.kernel(
      _body,
      out_type=jax.ShapeDtypeStruct((N_EDGES,), jnp.float32),
      mesh=mesh,
      compiler_params=pltpu.CompilerParams(
          needs_layout_passes=False, use_tc_tiling_on_sc=False),
      scratch_types=[
          pltpu.VMEM((E_PER,), jnp.int32),
          pltpu.VMEM((E_PER,), jnp.int32),
          pltpu.VMEM((E_PER,), jnp.float32),
          pltpu.VMEM((NP,), jnp.int32),
          pltpu.VMEM((CP, N_NODES), jnp.int32),
          pltpu.VMEM((CP, N_NODES), jnp.int32),
          pltpu.SemaphoreType.DMA,
          pltpu.SemaphoreType.DMA,
      ],
  )
  return f(ct, sidx, didx, w)


def kernel(sparse_codes, edge_index, pattern_weights):
  eidx = edge_index.astype(jnp.int32)
  codes_bf = sparse_codes.astype(jnp.bfloat16)
  ct = jax.lax.bitcast_convert_type(
      codes_bf.T.reshape(NP, 2, N_NODES).transpose(0, 2, 1), jnp.int32)
  w_bf = pattern_weights.astype(jnp.bfloat16)
  w_i32 = jax.lax.bitcast_convert_type(w_bf.reshape(NP, 2), jnp.int32)
  return _run(ct, eidx[0], eidx[1], w_i32)


# speedup vs baseline: 1.2315x; 1.2315x over previous
import functools

import jax
import jax.numpy as jnp
from jax import lax
from jax.experimental import pallas as pl
from jax.experimental.pallas import tpu as pltpu
from jax.experimental.pallas import tpu_sc as plsc

N_NODES = 10000
N_EDGES = 320000
A = 128
L = 16
LB = 2 * L
NC = 2
NS = 16
NW = NC * NS
E_PER = N_EDGES // NW
K = 80
NBLK = E_PER // K
NGRP = K // L
NJ = A // LB
AW = A // 2


def _body(codes_hbm, sidx_hbm, didx_hbm, w_hbm, out_hbm,
          si_v, di_v, s0, s1, d0, d1, ost, wv, bt, table_sh,
          ss0, ss1, ds0, ds1):
  cid = lax.axis_index("c")
  sid = lax.axis_index("s")
  wid = sid * NC + cid
  base = wid * E_PER

  rows0 = sid * (N_NODES // NS)
  for t in range(7):
    pltpu.sync_copy(codes_hbm.at[pl.ds(rows0 + t * K, K)], s0)
    pltpu.sync_copy(s0, table_sh.at[pl.ds(rows0 + t * K, K)])
  tail = N_NODES // NS - 7 * K
  pltpu.sync_copy(codes_hbm.at[pl.ds(rows0 + 7 * K, tail)],
                  s0.at[pl.ds(0, tail)])
  pltpu.sync_copy(s0.at[pl.ds(0, tail)],
                  table_sh.at[pl.ds(rows0 + 7 * K, tail)])

  pltpu.sync_copy(sidx_hbm.at[pl.ds(base, E_PER)], si_v)
  pltpu.sync_copy(didx_hbm.at[pl.ds(base, E_PER)], di_v)
  pltpu.sync_copy(w_hbm, wv)
  plsc.subcore_barrier()

  sbuf = (s0, s1)
  dbuf = (d0, d1)
  ssem = (ss0, ss1)
  dsem = (ds0, ds1)

  def start_blk(g, b):
    i0 = g * K
    pltpu.async_copy(table_sh.at[si_v.at[pl.ds(i0, K)]], sbuf[b], ssem[b])
    pltpu.async_copy(table_sh.at[di_v.at[pl.ds(i0, K)]], dbuf[b], dsem[b])

  def wait_blk(g, b):
    i0 = g * K
    pltpu.make_async_copy(
        table_sh.at[si_v.at[pl.ds(i0, K)]], sbuf[b], ssem[b]).wait()
    pltpu.make_async_copy(
        table_sh.at[di_v.at[pl.ds(i0, K)]], dbuf[b], dsem[b]).wait()

  lane = lax.iota(jnp.int32, L)

  def compute_blk(g, b):
    srows = sbuf[b]
    drows = dbuf[b]

    @pl.loop(0, NGRP)
    def _grp(grp):
      wregs = [wv[pl.ds(j * LB, LB)] for j in range(NJ)]
      e0 = g * K + grp * L

      def row_bf(rows, e, j):
        return plsc.bitcast(rows[e, pl.ds(j * L, L)], jnp.bfloat16)

      for k in range(L):
        e = grp * L + k
        accp = row_bf(srows, e, 0) * row_bf(drows, e, 0) * wregs[0]
        for j in range(1, NJ):
          accp = jnp.maximum(
              accp, row_bf(srows, e, j) * row_bf(drows, e, j) * wregs[j])
        lo, hi = plsc.unpack(
            accp, format=plsc.PackFormat.INTERLEAVED,
            preferred_element_type=jnp.float32)
        acc = jnp.maximum(lo, hi)
        plsc.store_scatter(bt, [lane * L + k], acc)
      res = bt[pl.ds(0, L)]
      for l in range(1, L):
        res = jnp.maximum(res, bt[pl.ds(l * L, L)])
      ost[pl.ds(e0, L)] = res

  start_blk(0, 0)
  start_blk(1, 1)

  @pl.loop(0, (NBLK + 1) // 2)
  def _outer(gg):
    for b in range(2):
      g = gg * 2 + b

      @pl.when(g < NBLK)
      def _():
        wait_blk(g, b)

        @pl.when(g + 2 < NBLK)
        def _():
          start_blk(g + 2, b)

        compute_blk(g, b)

  @pl.loop(0, E_PER // L)
  def _sig(i):
    x = ost[pl.ds(i * L, L)]
    ost[pl.ds(i * L, L)] = 1.0 / (1.0 + jnp.exp(-x))

  pltpu.sync_copy(ost, out_hbm.at[pl.ds(base, E_PER)])


@jax.jit
def _run(codes, sidx, didx, w):
  mesh = plsc.VectorSubcoreMesh(
      core_axis_name="c", subcore_axis_name="s", num_cores=NC,
      num_subcores=NS)
  f = pl.kernel(
      _body,
      out_type=jax.ShapeDtypeStruct((N_EDGES,), jnp.float32),
      mesh=mesh,
      compiler_params=pltpu.CompilerParams(
          needs_layout_passes=False, use_tc_tiling_on_sc=False),
      scratch_types=[
          pltpu.VMEM((E_PER,), jnp.int32),
          pltpu.VMEM((E_PER,), jnp.int32),
          pltpu.VMEM((K, AW), jnp.int32),
          pltpu.VMEM((K, AW), jnp.int32),
          pltpu.VMEM((K, AW), jnp.int32),
          pltpu.VMEM((K, AW), jnp.int32),
          pltpu.VMEM((E_PER,), jnp.float32),
          pltpu.VMEM((A,), jnp.bfloat16),
          pltpu.VMEM((L * L,), jnp.float32),
          pltpu.VMEM_SHARED((N_NODES, AW), jnp.int32),
          pltpu.SemaphoreType.DMA,
          pltpu.SemaphoreType.DMA,
          pltpu.SemaphoreType.DMA,
          pltpu.SemaphoreType.DMA,
      ],
  )
  return f(codes, sidx, didx, w)


def kernel(sparse_codes, edge_index, pattern_weights):
  eidx = edge_index.astype(jnp.int32)
  codes_bf = sparse_codes.astype(jnp.bfloat16)
  codes_i32 = jax.lax.bitcast_convert_type(
      codes_bf.reshape(N_NODES, AW, 2), jnp.int32)
  w_bf = pattern_weights.astype(jnp.bfloat16)
  return _run(codes_i32, eidx[0], eidx[1], w_bf)


# speedup vs baseline: 1.7187x; 1.3956x over previous
import functools

import jax
import jax.numpy as jnp
from jax import lax
from jax.experimental import pallas as pl
from jax.experimental.pallas import tpu as pltpu
from jax.experimental.pallas import tpu_sc as plsc

N_NODES = 10000
N_EDGES = 320000
A = 128
L = 16
NP = A // 2
CP = 4
NCHUNK = NP // CP
NC = 2
NS = 16
NW = NC * NS
E_PER = N_EDGES // NW
NG = E_PER // L


def _body(ct_hbm, sidx_hbm, didx_hbm, w_hbm, out_hbm,
          si_v, di_v, pmax, wv, sl0, sl1, sem0, sem1):
  cid = lax.axis_index("c")
  sid = lax.axis_index("s")
  wid = sid * NC + cid
  base = wid * E_PER

  pltpu.sync_copy(sidx_hbm.at[pl.ds(base, E_PER)], si_v)
  pltpu.sync_copy(didx_hbm.at[pl.ds(base, E_PER)], di_v)
  pltpu.sync_copy(w_hbm, wv)

  slab = (sl0, sl1)
  sem = (sem0, sem1)

  def start_chunk(c, b):
    pltpu.async_copy(ct_hbm.at[pl.ds(c * CP, CP)], slab[b], sem[b])

  def wait_chunk(c, b):
    pltpu.make_async_copy(ct_hbm.at[pl.ds(c * CP, CP)], slab[b], sem[b]).wait()

  start_chunk(0, 0)
  start_chunk(1, 1)

  for c in range(NCHUNK):
    b = c % 2
    wait_chunk(c, b)
    if c + 2 < NCHUNK:
      start_chunk(c + 2, b)
    sl = slab[b]
    wwin = wv[pl.ds((c * CP // L) * L, L)]
    woff = c * CP - (c * CP // L) * L
    wp = [
        plsc.bitcast(jnp.full((L,), wwin[woff + jj], jnp.int32),
                     jnp.bfloat16)
        for jj in range(CP)
    ]
    first = c == 0

    @plsc.parallel_loop(0, NG)
    def _grp(grp, sl=sl, wp=wp, first=first):
      sv = si_v[pl.ds(grp * L, L)]
      dv = di_v[pl.ds(grp * L, L)]
      accp = None
      for jj in range(CP):
        row = sl.at[jj]
        s = plsc.bitcast(plsc.load_gather(row, [sv]), jnp.bfloat16)
        d = plsc.bitcast(plsc.load_gather(row, [dv]), jnp.bfloat16)
        m = s * d * wp[jj]
        accp = m if jj == 0 else jnp.maximum(accp, m)
      lo, hi = plsc.unpack(
          accp, format=plsc.PackFormat.INTERLEAVED,
          preferred_element_type=jnp.float32)
      cm = jnp.maximum(lo, hi)
      if not first:
        cm = jnp.maximum(cm, pmax[pl.ds(grp * L, L)])
      pmax[pl.ds(grp * L, L)] = cm

  @plsc.parallel_loop(0, NG)
  def _sig(i):
    x = pmax[pl.ds(i * L, L)]
    pmax[pl.ds(i * L, L)] = 1.0 / (1.0 + jnp.exp(-x))

  pltpu.sync_copy(pmax, out_hbm.at[pl.ds(base, E_PER)])


@jax.jit
def _run(ct, sidx, didx, w):
  mesh = plsc.VectorSubcoreMesh(
      core_axis_name="c", subcore_axis_name="s", num_cores=NC,
      num_subcores=NS)
  f = pl.kernel(
      _body,
      out_type=jax.ShapeDtypeStruct((N_EDGES,), jnp.float32),
      mesh=mesh,
      compiler_params=pltpu.CompilerParams(
          needs_layout_passes=False, use_tc_tiling_on_sc=False),
      scratch_types=[
          pltpu.VMEM((E_PER,), jnp.int32),
          pltpu.VMEM((E_PER,), jnp.int32),
          pltpu.VMEM((E_PER,), jnp.float32),
          pltpu.VMEM((NP,), jnp.int32),
          pltpu.VMEM((CP, N_NODES), jnp.int32),
          pltpu.VMEM((CP, N_NODES), jnp.int32),
          pltpu.SemaphoreType.DMA,
          pltpu.SemaphoreType.DMA,
      ],
  )
  return f(ct, sidx, didx, w)


def kernel(sparse_codes, edge_index, pattern_weights):
  eidx = edge_index.astype(jnp.int32)
  codes_bf = sparse_codes.astype(jnp.bfloat16)
  ct = jax.lax.bitcast_convert_type(
      codes_bf.T.reshape(NP, 2, N_NODES).transpose(0, 2, 1), jnp.int32)
  w_bf = pattern_weights.astype(jnp.bfloat16)
  w_i32 = jax.lax.bitcast_convert_type(w_bf.reshape(NP, 2), jnp.int32)
  return _run(ct, eidx[0], eidx[1], w_i32)


# speedup vs baseline: 1.7452x; 1.0154x over previous
import functools

import jax
import jax.numpy as jnp
from jax import lax
from jax.experimental import pallas as pl
from jax.experimental.pallas import tpu as pltpu
from jax.experimental.pallas import tpu_sc as plsc

N_NODES = 10000
N_EDGES = 320000
A = 128
L = 16
NP = A // 2
CP = 4
NCHUNK = NP // CP
NC = 2
NS = 16
NW = NC * NS
E_PER = N_EDGES // NW
NG = E_PER // L


def _body(ct_hbm, sidx_hbm, didx_hbm, w_hbm, out_hbm,
          si_v, di_v, pmax, wv, sl0, sl1, sem0, sem1):
  cid = lax.axis_index("c")
  sid = lax.axis_index("s")
  wid = sid * NC + cid
  base = wid * E_PER

  pltpu.sync_copy(sidx_hbm.at[pl.ds(base, E_PER)], si_v)
  pltpu.sync_copy(didx_hbm.at[pl.ds(base, E_PER)], di_v)
  pltpu.sync_copy(w_hbm, wv)

  slab = (sl0, sl1)
  sem = (sem0, sem1)

  def start_chunk(c, b):
    pltpu.async_copy(ct_hbm.at[pl.ds(c * CP, CP)], slab[b], sem[b])

  def wait_chunk(c, b):
    pltpu.make_async_copy(ct_hbm.at[pl.ds(c * CP, CP)], slab[b], sem[b]).wait()

  start_chunk(0, 0)
  start_chunk(1, 1)

  for c in range(NCHUNK):
    b = c % 2
    wait_chunk(c, b)
    if c + 2 < NCHUNK:
      start_chunk(c + 2, b)
    sl = slab[b]
    wwin = wv[pl.ds((c * CP // L) * L, L)]
    woff = c * CP - (c * CP // L) * L
    wp = [
        plsc.bitcast(jnp.full((L,), wwin[woff + jj], jnp.int32),
                     jnp.bfloat16)
        for jj in range(CP)
    ]
    first = c == 0

    @plsc.parallel_loop(0, NG, unroll=2)
    def _grp(grp, sl=sl, wp=wp, first=first):
      sv = si_v[pl.ds(grp * L, L)]
      dv = di_v[pl.ds(grp * L, L)]
      accp = None
      for jj in range(CP):
        row = sl.at[jj]
        s = plsc.bitcast(plsc.load_gather(row, [sv]), jnp.bfloat16)
        d = plsc.bitcast(plsc.load_gather(row, [dv]), jnp.bfloat16)
        m = s * d * wp[jj]
        accp = m if jj == 0 else jnp.maximum(accp, m)
      lo, hi = plsc.unpack(
          accp, format=plsc.PackFormat.INTERLEAVED,
          preferred_element_type=jnp.float32)
      cm = jnp.maximum(lo, hi)
      if not first:
        cm = jnp.maximum(cm, pmax[pl.ds(grp * L, L)])
      pmax[pl.ds(grp * L, L)] = cm

  @plsc.parallel_loop(0, NG)
  def _sig(i):
    x = pmax[pl.ds(i * L, L)]
    pmax[pl.ds(i * L, L)] = 1.0 / (1.0 + jnp.exp(-x))

  pltpu.sync_copy(pmax, out_hbm.at[pl.ds(base, E_PER)])


@jax.jit
def _run(ct, sidx, didx, w):
  mesh = plsc.VectorSubcoreMesh(
      core_axis_name="c", subcore_axis_name="s", num_cores=NC,
      num_subcores=NS)
  f = pl.kernel(
      _body,
      out_type=jax.ShapeDtypeStruct((N_EDGES,), jnp.float32),
      mesh=mesh,
      compiler_params=pltpu.CompilerParams(
          needs_layout_passes=False, use_tc_tiling_on_sc=False),
      scratch_types=[
          pltpu.VMEM((E_PER,), jnp.int32),
          pltpu.VMEM((E_PER,), jnp.int32),
          pltpu.VMEM((E_PER,), jnp.float32),
          pltpu.VMEM((NP,), jnp.int32),
          pltpu.VMEM((CP, N_NODES), jnp.int32),
          pltpu.VMEM((CP, N_NODES), jnp.int32),
          pltpu.SemaphoreType.DMA,
          pltpu.SemaphoreType.DMA,
      ],
  )
  return f(ct, sidx, didx, w)


def kernel(sparse_codes, edge_index, pattern_weights):
  eidx = edge_index.astype(jnp.int32)
  codes_bf = sparse_codes.astype(jnp.bfloat16)
  ct = jax.lax.bitcast_convert_type(
      codes_bf.T.reshape(NP, 2, N_NODES).transpose(0, 2, 1), jnp.int32)
  w_bf = pattern_weights.astype(jnp.bfloat16)
  w_i32 = jax.lax.bitcast_convert_type(w_bf.reshape(NP, 2), jnp.int32)
  return _run(ct, eidx[0], eidx[1], w_i32)


# speedup vs baseline: 2.4132x; 1.3828x over previous
import functools

import jax
import jax.numpy as jnp
from jax import lax
from jax.experimental import pallas as pl
from jax.experimental.pallas import tpu as pltpu
from jax.experimental.pallas import tpu_sc as plsc

N_NODES = 10000
N_EDGES = 320000
A = 128
L = 16
NQ = A // 4
CQ = 4
NCHUNK = NQ // CQ
NC = 2
NS = 16
NW = NC * NS
E_PER = N_EDGES // NW
NG = E_PER // L


def _body(ct_hbm, sidx_hbm, didx_hbm, w_hbm, out_hbm,
          si_v, di_v, pmax, wv, sl0, sl1, sem0, sem1):
  cid = lax.axis_index("c")
  sid = lax.axis_index("s")
  wid = sid * NC + cid
  base = wid * E_PER

  pltpu.sync_copy(sidx_hbm.at[pl.ds(base, E_PER)], si_v)
  pltpu.sync_copy(didx_hbm.at[pl.ds(base, E_PER)], di_v)
  pltpu.sync_copy(w_hbm, wv)

  slab = (sl0, sl1)
  sem = (sem0, sem1)

  def start_chunk(c, b):
    pltpu.async_copy(ct_hbm.at[pl.ds(c * CQ, CQ)], slab[b], sem[b])

  def wait_chunk(c, b):
    pltpu.make_async_copy(ct_hbm.at[pl.ds(c * CQ, CQ)], slab[b], sem[b]).wait()

  start_chunk(0, 0)
  start_chunk(1, 1)

  for c in range(NCHUNK):
    b = c % 2
    wait_chunk(c, b)
    if c + 2 < NCHUNK:
      start_chunk(c + 2, b)
    sl = slab[b]
    wwin = wv[pl.ds((c // 2) * L, L)]
    woff = 2 * CQ * (c % 2)
    wpa = [
        plsc.bitcast(jnp.full((L,), wwin[woff + 2 * qq], jnp.int32),
                     jnp.bfloat16)
        for qq in range(CQ)
    ]
    wpb = [
        plsc.bitcast(jnp.full((L,), wwin[woff + 2 * qq + 1], jnp.int32),
                     jnp.bfloat16)
        for qq in range(CQ)
    ]
    first = c == 0

    @plsc.parallel_loop(0, NG, unroll=2)
    def _grp(grp, sl=sl, wpa=wpa, wpb=wpb, first=first):
      sv = si_v[pl.ds(grp * L, L)]
      dv = di_v[pl.ds(grp * L, L)]
      accp = None
      for qq in range(CQ):
        row = sl.at[qq]
        fs = plsc.bitcast(plsc.load_gather(row, [sv]), jnp.float8_e4m3fn)
        fd = plsc.bitcast(plsc.load_gather(row, [dv]), jnp.float8_e4m3fn)
        sa, sb = plsc.unpack(
            fs, format=plsc.PackFormat.INTERLEAVED,
            preferred_element_type=jnp.bfloat16)
        da, db = plsc.unpack(
            fd, format=plsc.PackFormat.INTERLEAVED,
            preferred_element_type=jnp.bfloat16)
        m = jnp.maximum(sa * da * wpa[qq], sb * db * wpb[qq])
        accp = m if qq == 0 else jnp.maximum(accp, m)
      lo, hi = plsc.unpack(
          accp, format=plsc.PackFormat.INTERLEAVED,
          preferred_element_type=jnp.float32)
      cm = jnp.maximum(lo, hi)
      if not first:
        cm = jnp.maximum(cm, pmax[pl.ds(grp * L, L)])
      pmax[pl.ds(grp * L, L)] = cm

  @plsc.parallel_loop(0, NG)
  def _sig(i):
    x = pmax[pl.ds(i * L, L)]
    pmax[pl.ds(i * L, L)] = 1.0 / (1.0 + jnp.exp(-x))

  pltpu.sync_copy(pmax, out_hbm.at[pl.ds(base, E_PER)])


@jax.jit
def _run(ct, sidx, didx, w):
  mesh = plsc.VectorSubcoreMesh(
      core_axis_name="c", subcore_axis_name="s", num_cores=NC,
      num_subcores=NS)
  f = pl.kernel(
      _body,
      out_type=jax.ShapeDtypeStruct((N_EDGES,), jnp.float32),
      mesh=mesh,
      compiler_params=pltpu.CompilerParams(
          needs_layout_passes=False, use_tc_tiling_on_sc=False),
      scratch_types=[
          pltpu.VMEM((E_PER,), jnp.int32),
          pltpu.VMEM((E_PER,), jnp.int32),
          pltpu.VMEM((E_PER,), jnp.float32),
          pltpu.VMEM((2 * NQ,), jnp.int32),
          pltpu.VMEM((CQ, N_NODES), jnp.int32),
          pltpu.VMEM((CQ, N_NODES), jnp.int32),
          pltpu.SemaphoreType.DMA,
          pltpu.SemaphoreType.DMA,
      ],
  )
  return f(ct, sidx, didx, w)


def kernel(sparse_codes, edge_index, pattern_weights):
  eidx = edge_index.astype(jnp.int32)
  codes_f8 = sparse_codes.astype(jnp.float8_e4m3fn)
  ct = jax.lax.bitcast_convert_type(
      codes_f8.T.reshape(NQ, 4, N_NODES).transpose(0, 2, 1), jnp.int32)
  w_bf = pattern_weights.astype(jnp.bfloat16)
  w_i32 = jax.lax.bitcast_convert_type(
      w_bf.reshape(NQ, 2, 2).transpose(0, 2, 1), jnp.int32).reshape(2 * NQ)
  return _run(ct, eidx[0], eidx[1], w_i32)
